# Initial kernel scaffold; baseline (speedup 1.0000x reference)
import jax
import jax.numpy as jnp
from jax.experimental import pallas as pl


def kernel(input_bert_features, attention_mask, spans, span_mask, related_spans_tensor, sentence_length, Wi_f, bi_f, Wo_f, bo_f, g_f, be_f, Wi_r, bi_r, Wo_r, bo_r, g_r, be_r, Wa, ba, Wop, bop):
    raise NotImplementedError("write your pallas kernel here")


# speedup vs baseline: 7.9480x; 7.9480x over previous
import functools

import jax
import jax.numpy as jnp
from jax.experimental import pallas as pl
from jax.experimental.pallas import tpu as pltpu

B, S, D = 8, 512, 768
SPAN_NUM = 2048
MAX_W = 4
D_FF = 3072
N_CLS = 3

M_TILE = 512
NT = SPAN_NUM // M_TILE
LANES = 128


def _layer_norm(y, g, b, eps=1e-12):
    m = jnp.mean(y, axis=-1, keepdims=True)
    c = y - m
    v = jnp.mean(c * c, axis=-1, keepdims=True)
    return c * jax.lax.rsqrt(v + eps) * g + b


def _gelu_exact(x):
    return 0.5 * x * (1.0 + jax.lax.erf(x * 0.7071067811865476))


def _fused_body(p_ref, x_ref, wi_f, bi_f, wo_f, bo_f, g_f, be_f,
                wi_r, bi_r, wo_r, bo_r, g_r, be_r, wab, bab, out_ref):
    p = p_ref[0, 0]
    start = p[0:1, :]
    end = p[1:2, :]
    invw = p[2:3, :]
    xb = x_ref[0]


    s_iota = jax.lax.broadcasted_iota(jnp.int32, (S, M_TILE), 0).astype(jnp.float32)
    sel = jnp.logical_and(s_iota >= start, s_iota < end)
    at = jnp.where(sel, invw, 0.0)

    e = jax.lax.dot_general(at, xb, (((0,), (0,)), ((), ())),
                            preferred_element_type=jnp.float32)

    def decoder(wi, bi, wo, bo, g, be):
        h = jnp.dot(e, wi[...], preferred_element_type=jnp.float32) + bi[...]
        h = _gelu_exact(h)
        o = jnp.dot(h, wo[...], preferred_element_type=jnp.float32) + bo[...]
        return _layer_norm(o + e, g[...], be[...])

    ln_f = decoder(wi_f, bi_f, wo_f, bo_f, g_f, be_f)
    ln_r = decoder(wi_r, bi_r, wo_r, bo_r, g_r, be_r)

    logits = (jnp.dot(ln_f, wab[0], preferred_element_type=jnp.float32)
              + jnp.dot(ln_r, wab[1], preferred_element_type=jnp.float32)
              + bab[...])
    out_ref[...] = logits


@jax.jit
def _fused(p, x, wi_f, bi_f, wo_f, bo_f, g_f, be_f,
           wi_r, bi_r, wo_r, bo_r, g_r, be_r, wab, bab):
    full = lambda shape: pl.BlockSpec(shape, lambda b, t: (0,) * len(shape))
    grid = (B, NT)
    return pl.pallas_call(
        _fused_body,
        grid=grid,
        in_specs=[
            pl.BlockSpec((1, 1, 8, M_TILE), lambda b, t: (b, t, 0, 0)),
            pl.BlockSpec((1, S, D), lambda b, t: (b, 0, 0)),
            full((D, D_FF)), full((1, D_FF)), full((D_FF, D)), full((1, D)),
            full((1, D)), full((1, D)),
            full((D, D_FF)), full((1, D_FF)), full((D_FF, D)), full((1, D)),
            full((1, D)), full((1, D)),
            full((2, D, LANES)), full((1, LANES)),
        ],
        out_specs=pl.BlockSpec((M_TILE, LANES), lambda b, t: (b * NT + t, 0)),
        out_shape=jax.ShapeDtypeStruct((B * SPAN_NUM, LANES), jnp.float32),
        compiler_params=pltpu.CompilerParams(
            dimension_semantics=("parallel", "parallel")),
    )(p, x, wi_f, bi_f, wo_f, bo_f, g_f, be_f,
      wi_r, bi_r, wo_r, bo_r, g_r, be_r, wab, bab)


def kernel(input_bert_features, attention_mask, spans, span_mask,
           related_spans_tensor, sentence_length, Wi_f, bi_f, Wo_f, bo_f,
           g_f, be_f, Wi_r, bi_r, Wo_r, bo_r, g_r, be_r, Wa, ba, Wop, bop):
    start = spans[..., 0].astype(jnp.float32)
    width = spans[..., 2].astype(jnp.float32)
    end = start + width
    invw = span_mask.astype(jnp.float32) / jnp.maximum(width, 1.0)
    pack = jnp.stack([start, end, invw], axis=-2)

    p = jnp.zeros((B, 8, SPAN_NUM), jnp.float32).at[:, :3, :].set(pack)
    p = p.reshape(B, 8, NT, M_TILE).transpose(0, 2, 1, 3)

    wab = jnp.zeros((2, D, LANES), jnp.float32)
    wab = wab.at[0, :, :N_CLS].set(Wa).at[1, :, N_CLS:2 * N_CLS].set(Wop)
    bab = jnp.zeros((1, LANES), jnp.float32)
    bab = bab.at[0, :N_CLS].set(ba).at[0, N_CLS:2 * N_CLS].set(bop)

    out = _fused(p, input_bert_features,
                 Wi_f, bi_f.reshape(1, D_FF), Wo_f, bo_f.reshape(1, D),
                 g_f.reshape(1, D), be_f.reshape(1, D),
                 Wi_r, bi_r.reshape(1, D_FF), Wo_r, bo_r.reshape(1, D),
                 g_r.reshape(1, D), be_r.reshape(1, D), wab, bab)
    return out[:, :2 * N_CLS].reshape(B, SPAN_NUM, 2 * N_CLS)
